# baseline (speedup 1.0000x reference)
import functools

import jax
import jax.numpy as jnp
from jax import lax
from jax.experimental import pallas as pl
from jax.experimental.pallas import tpu as pltpu
from jax.experimental.pallas import tpu_sc as plsc

_NC = 2
_NS = 16
_NW = _NC * _NS
_L = 16
_BT = 128


def _body(
    h,
    d,
    dp,
    clip,
    time_hbm,
    table_hbm,
    out_hbm,
    table_v,
    t_v,
    oba0,
    obb0,
    oba1,
    obb1,
    obd,
    sa0,
    sb0,
    sa1,
    sb1,
    sd,
):
    wid = lax.axis_index("s") * _NC + lax.axis_index("c")
    n_g = _BT // _L

    pltpu.sync_copy(table_hbm, table_v)
    pltpu.sync_copy(time_hbm.at[:, pl.ds(wid * _BT, _BT)], t_v)

    t0 = [table_v[pl.ds(0, _L)], table_v[pl.ds(_L, _L)]]
    for c in range(d):
        v = jnp.full((_L,), t0[c // _L][c % _L], jnp.float32)
        for g in range(n_g):
            obd[c // 8, c % 8, pl.ds(g * _L, _L)] = v

    for i in range(h):
        pltpu.async_copy(obd, out_hbm.at[i * h + i, :, wid, :, :], sd)

    def unit(i, q, oba, obb, sema, semb, first):
        @pl.when(jnp.logical_not(first))
        def _():
            pltpu.make_async_copy(
                oba, out_hbm.at[0, :, wid, :, :], sema
            ).wait()
            pltpu.make_async_copy(
                obb, out_hbm.at[0, :, wid, :, :], semb
            ).wait()

        @plsc.parallel_loop(0, n_g, unroll=1)
        def grp(g):
            gl = g * _L
            ti = t_v[i, pl.ds(gl, _L)]
            tq = t_v[q, pl.ds(gl, _L)]
            rows16 = jnp.minimum(jnp.abs(ti - tq), clip)
            wb = rows16 * dp
            for c in range(d):
                v = plsc.load_gather(table_v, [wb + c])
                oba[c // 8, c % 8, pl.ds(gl, _L)] = v
                obb[c // 8, c % 8, pl.ds(gl, _L)] = v

        pltpu.async_copy(oba, out_hbm.at[i * h + q, :, wid, :, :], sema)
        pltpu.async_copy(obb, out_hbm.at[q * h + i, :, wid, :, :], semb)

    def qbody(i, q, u):
        @pl.when(u % 2 == 0)
        def _():
            unit(i, q, oba0, obb0, sa0, sb0, u < 2)

        @pl.when(u % 2 == 1)
        def _():
            unit(i, q, oba1, obb1, sa1, sb1, u < 2)

        return u + 1

    def ibody(i, u):
        return lax.fori_loop(i + 1, h, functools.partial(qbody, i), u)

    u = lax.fori_loop(0, h - 1, ibody, jnp.int32(0))

    @pl.when(u >= 2)
    def _():
        pltpu.make_async_copy(oba0, out_hbm.at[0, :, wid, :, :], sa0).wait()
        pltpu.make_async_copy(obb0, out_hbm.at[0, :, wid, :, :], sb0).wait()

    @pl.when(u >= 1)
    def _():
        pltpu.make_async_copy(oba1, out_hbm.at[0, :, wid, :, :], sa1).wait()
        pltpu.make_async_copy(obb1, out_hbm.at[0, :, wid, :, :], sb1).wait()

    for _i in range(h):
        pltpu.make_async_copy(obd, out_hbm.at[0, :, wid, :, :], sd).wait()


def kernel(time, table, max_len):
    b, h = time.shape
    v, d = table.shape
    clip = v - 1

    assert d % 8 == 0 and _BT % _L == 0
    nbt = b // _BT
    assert nbt == _NW
    nct = d // 8

    dp = d + 1
    table_pad = jnp.concatenate(
        [table, jnp.zeros((v, 1), jnp.float32)], axis=1
    ).reshape(v * dp)

    mesh = plsc.VectorSubcoreMesh(core_axis_name="c", subcore_axis_name="s")
    blk = (nct, 8, _BT)
    out = pl.kernel(
        functools.partial(_body, h, d, dp, clip),
        out_type=jax.ShapeDtypeStruct((h * h, nct, nbt, 8, _BT), jnp.float32),
        mesh=mesh,
        scratch_types=[
            pltpu.VMEM((v * dp,), jnp.float32),
            pltpu.VMEM((h, _BT), jnp.int32),
            pltpu.VMEM(blk, jnp.float32),
            pltpu.VMEM(blk, jnp.float32),
            pltpu.VMEM(blk, jnp.float32),
            pltpu.VMEM(blk, jnp.float32),
            pltpu.VMEM(blk, jnp.float32),
            pltpu.SemaphoreType.DMA,
            pltpu.SemaphoreType.DMA,
            pltpu.SemaphoreType.DMA,
            pltpu.SemaphoreType.DMA,
            pltpu.SemaphoreType.DMA,
        ],
        compiler_params=pltpu.CompilerParams(
            use_tc_tiling_on_sc=False, needs_layout_passes=False
        ),
    )(time.T, table_pad)
    phys = out.reshape(h, h, nct, nbt, 8, _BT)
    res = phys.transpose(3, 5, 0, 1, 2, 4)
    return res.reshape(b, h, h, d)
